# baseline (speedup 1.0000x reference)
import jax
import jax.numpy as jnp
from jax.experimental import pallas as pl
from jax.experimental.pallas import tpu as pltpu

_MAX_CTX = 32768
_D = 1024
_NBUF = 2
_CHUNK_ROWS = 4096
_NCHUNKS = _MAX_CTX // _CHUNK_ROWS
assert _NCHUNKS % _NBUF == 0


def _copy_kernel(src_ref, dst_ref, buf, rsem, wsem):
    ngroups = _NCHUNKS // _NBUF
    for g in range(ngroups):
        for b in range(_NBUF):
            c = g * _NBUF + b
            if g > 0:
                pltpu.make_async_copy(
                    buf.at[b], dst_ref.at[pl.ds((c - _NBUF) * _CHUNK_ROWS, _CHUNK_ROWS), :], wsem.at[b]
                ).wait()
            pltpu.make_async_copy(
                src_ref.at[pl.ds(c * _CHUNK_ROWS, _CHUNK_ROWS), :], buf.at[b], rsem.at[b]
            ).start()
        for b in range(_NBUF):
            c = g * _NBUF + b
            pltpu.make_async_copy(
                src_ref.at[pl.ds(c * _CHUNK_ROWS, _CHUNK_ROWS), :], buf.at[b], rsem.at[b]
            ).wait()
            pltpu.make_async_copy(
                buf.at[b], dst_ref.at[pl.ds(c * _CHUNK_ROWS, _CHUNK_ROWS), :], wsem.at[b]
            ).start()
    g = ngroups - 1
    for b in range(_NBUF):
        c = g * _NBUF + b
        pltpu.make_async_copy(
            buf.at[b], dst_ref.at[pl.ds(c * _CHUNK_ROWS, _CHUNK_ROWS), :], wsem.at[b]
        ).wait()


def kernel(h, mem):
    b, l, d = h.shape
    assert b * l == _MAX_CTX and d == _D
    flat = h.reshape(b * l, d)
    new_mem = pl.pallas_call(
        _copy_kernel,
        in_specs=[pl.BlockSpec(memory_space=pl.ANY)],
        out_specs=pl.BlockSpec(memory_space=pl.ANY),
        out_shape=jax.ShapeDtypeStruct((b * l, d), h.dtype),
        scratch_shapes=[
            pltpu.VMEM((_NBUF, _CHUNK_ROWS, _D), h.dtype),
            pltpu.SemaphoreType.DMA((_NBUF,)),
            pltpu.SemaphoreType.DMA((_NBUF,)),
        ],
    )(flat)
    return (h, new_mem)
